# Initial kernel scaffold; baseline (speedup 1.0000x reference)
import jax
import jax.numpy as jnp
from jax.experimental import pallas as pl


def kernel(emb, img_ids, Wa, ba, Wv, bv, Wo, bo):
    raise NotImplementedError("write your pallas kernel here")


# speedup vs baseline: 17.3420x; 17.3420x over previous
import functools

import jax
import jax.numpy as jnp
from jax import lax
from jax.experimental import pallas as pl
from jax.experimental.pallas import tpu as pltpu
from jax.experimental.pallas import tpu_sc as plsc

_B = 320000
_E = 128
_NSEG = 10000
_L = 16
_NW = 16
_CHUNK = _B // _NW
_SSEG = 640
_NSEGP = _NW * _SSEG
_TILE = 4000
assert _B % _TILE == 0


def _tc_body(emb_ref, wa_ref, wv_ref, wo_ref, st_ref):
    e = emb_ref[...]
    wa = wa_ref[0, :]
    weff = jnp.sum(wo_ref[0, :][:, None] * wv_ref[...], axis=0)
    col = jax.lax.broadcasted_iota(jnp.int32, (_E, 8), 1)
    k = jnp.where(col == 0, wa[:, None],
                  jnp.where(col == 1, weff[:, None], 0.0))
    st_ref[...] = jax.lax.dot_general(
        e, k, (((1,), (0,)), ((), ())),
        preferred_element_type=jnp.float32)


def _tc_stage(emb, Wa, Wv, Wo):
    n = _B // _TILE
    return pl.pallas_call(
        _tc_body,
        grid=(n,),
        in_specs=[
            pl.BlockSpec((_TILE, _E), lambda i: (i, 0)),
            pl.BlockSpec((1, _E), lambda i: (0, 0)),
            pl.BlockSpec((_E, _E), lambda i: (0, 0)),
            pl.BlockSpec((1, _E), lambda i: (0, 0)),
        ],
        out_specs=pl.BlockSpec((_TILE, 8), lambda i: (i, 0)),
        out_shape=jax.ShapeDtypeStruct((_B, 8), jnp.float32),
    )(emb, Wa, Wv, Wo)


_MESH = plsc.VectorSubcoreMesh(
    core_axis_name="c", subcore_axis_name="s", num_cores=1, num_subcores=_NW
)


def _sc_body(s_hbm, t_hbm, ids_hbm, cpb_hbm, bo_hbm,
             r_hbm, den_hbm, refl_hbm,
             sv, tv, iv, num_acc, den_acc, rtab, tmp, accn, accd,
             cvec, bvec, shn, shd, shr):
    wid = lax.axis_index("s")
    base = wid * _CHUNK

    pltpu.sync_copy(s_hbm.at[pl.ds(base, _CHUNK)], sv)
    pltpu.sync_copy(t_hbm.at[pl.ds(base, _CHUNK)], tv)
    pltpu.sync_copy(ids_hbm.at[pl.ds(base, _CHUNK)], iv)
    pltpu.sync_copy(cpb_hbm, cvec)
    pltpu.sync_copy(bo_hbm, bvec)

    zeros = jnp.zeros((_L,), jnp.float32)

    def zbody(j, c):
        sl = pl.ds(j * _L, _L)
        num_acc[sl] = zeros
        den_acc[sl] = zeros
        return c

    lax.fori_loop(0, _NSEGP // _L, zbody, 0)

    def sbody(i, c):
        sl = pl.ds(i * _L, _L)
        ex = jnp.exp(sv[sl])
        idx = iv[sl]
        plsc.addupdate_scatter(den_acc, [idx], ex)
        plsc.addupdate_scatter(num_acc, [idx], ex * tv[sl])
        return c

    lax.fori_loop(0, _CHUNK // _L, sbody, 0)

    pltpu.sync_copy(num_acc, shn.at[wid])
    pltpu.sync_copy(den_acc, shd.at[wid])
    plsc.subcore_barrier()

    off = wid * _SSEG

    def addloop(j, c):
        sl = pl.ds(j * _L, _L)
        accn[sl] = accn[sl] + tmp[sl]
        return c

    def addloop_d(j, c):
        sl = pl.ds(j * _L, _L)
        accd[sl] = accd[sl] + tmp[sl]
        return c

    pltpu.sync_copy(shn.at[0, pl.ds(off, _SSEG)], accn)
    pltpu.sync_copy(shd.at[0, pl.ds(off, _SSEG)], accd)
    for k in range(1, _NW):
        pltpu.sync_copy(shn.at[k, pl.ds(off, _SSEG)], tmp)
        lax.fori_loop(0, _SSEG // _L, addloop, 0)
        pltpu.sync_copy(shd.at[k, pl.ds(off, _SSEG)], tmp)
        lax.fori_loop(0, _SSEG // _L, addloop_d, 0)

    def fbody(j, c):
        sl = pl.ds(j * _L, _L)
        den = accd[sl]
        r = jnp.where(den > 0.0, accn[sl] / den + cvec[...], bvec[...])
        accn[sl] = r
        return c

    lax.fori_loop(0, _SSEG // _L, fbody, 0)

    pltpu.sync_copy(accn, r_hbm.at[pl.ds(off, _SSEG)])
    pltpu.sync_copy(accd, den_hbm.at[pl.ds(off, _SSEG)])
    pltpu.sync_copy(accn, shr.at[pl.ds(off, _SSEG)])
    plsc.subcore_barrier()

    pltpu.sync_copy(shr, rtab)

    def gbody(i, c):
        sl = pl.ds(i * _L, _L)
        sv[sl] = plsc.load_gather(rtab, [iv[sl]])
        return c

    lax.fori_loop(0, _CHUNK // _L, gbody, 0)
    pltpu.sync_copy(sv, refl_hbm.at[pl.ds(base, _CHUNK)])


_sc_stage = functools.partial(
    pl.kernel,
    out_type=[
        jax.ShapeDtypeStruct((_NSEGP,), jnp.float32),
        jax.ShapeDtypeStruct((_NSEGP,), jnp.float32),
        jax.ShapeDtypeStruct((_B,), jnp.float32),
    ],
    mesh=_MESH,
    compiler_params=pltpu.CompilerParams(needs_layout_passes=False),
    scratch_types=[
        pltpu.VMEM((_CHUNK,), jnp.float32),
        pltpu.VMEM((_CHUNK,), jnp.float32),
        pltpu.VMEM((_CHUNK,), jnp.int32),
        pltpu.VMEM((_NSEGP,), jnp.float32),
        pltpu.VMEM((_NSEGP,), jnp.float32),
        pltpu.VMEM((_NSEGP,), jnp.float32),
        pltpu.VMEM((_SSEG,), jnp.float32),
        pltpu.VMEM((_SSEG,), jnp.float32),
        pltpu.VMEM((_SSEG,), jnp.float32),
        pltpu.VMEM((_L,), jnp.float32),
        pltpu.VMEM((_L,), jnp.float32),
        pltpu.VMEM_SHARED((_NW, _NSEGP), jnp.float32),
        pltpu.VMEM_SHARED((_NW, _NSEGP), jnp.float32),
        pltpu.VMEM_SHARED((_NSEGP,), jnp.float32),
    ],
)(_sc_body)


def kernel(emb, img_ids, Wa, ba, Wv, bv, Wo, bo):
    del ba
    ids = img_ids.astype(jnp.int32)
    st = _tc_stage(emb, Wa, Wv, Wo)
    scores = st[:, 0]
    t = st[:, 1]
    cpb = jnp.dot(Wo[0], bv) + bo[0]
    cpb_vec = jnp.full((_L,), cpb, jnp.float32)
    bo_vec = jnp.full((_L,), bo[0], jnp.float32)
    r_pad, den_pad, refl = _sc_stage(scores, t, ids, cpb_vec, bo_vec)
    r_flat = r_pad[:_NSEG]
    den = den_pad[:_NSEG]
    occ = den > 0
    n = occ.sum()
    nz = jnp.nonzero(occ, size=_NSEG, fill_value=0)[0]
    unique_ids = jnp.where(jnp.arange(_NSEG) < n, nz, ids[0]).astype(img_ids.dtype)
    return refl[:, None], r_flat[:, None], unique_ids


# speedup vs baseline: 36.7619x; 2.1198x over previous
import functools

import jax
import jax.numpy as jnp
from jax import lax
from jax.experimental import pallas as pl
from jax.experimental.pallas import tpu as pltpu
from jax.experimental.pallas import tpu_sc as plsc

_B = 320000
_E = 128
_NSEG = 10000
_L = 16
_NW = 16
_CHUNK = _B // _NW
_SSEG = 640
_NSEGP = _NW * _SSEG
_TILE = 8192


def _tc_body(emb_ref, wa_ref, wv_ref, wo_ref, s_ref, t_ref):
    e = emb_ref[...]
    wa = wa_ref[0, :]
    weff = jnp.sum(wo_ref[0, :][:, None] * wv_ref[...], axis=0)
    row = jax.lax.broadcasted_iota(jnp.int32, (8, _E), 0)
    k = jnp.where(row == 0, wa[None, :],
                  jnp.where(row == 1, weff[None, :], 0.0))
    st8 = jax.lax.dot_general(
        k, e, (((1,), (1,)), ((), ())),
        preferred_element_type=jnp.float32)
    s_ref[...] = st8[0, :]
    t_ref[...] = st8[1, :]


def _tc_stage(emb, Wa, Wv, Wo):
    n = pl.cdiv(_B, _TILE)
    return pl.pallas_call(
        _tc_body,
        grid=(n,),
        in_specs=[
            pl.BlockSpec((_TILE, _E), lambda i: (i, 0)),
            pl.BlockSpec((1, _E), lambda i: (0, 0)),
            pl.BlockSpec((_E, _E), lambda i: (0, 0)),
            pl.BlockSpec((1, _E), lambda i: (0, 0)),
        ],
        out_specs=[
            pl.BlockSpec((_TILE,), lambda i: (i,)),
            pl.BlockSpec((_TILE,), lambda i: (i,)),
        ],
        out_shape=[
            jax.ShapeDtypeStruct((_B,), jnp.float32),
            jax.ShapeDtypeStruct((_B,), jnp.float32),
        ],
    )(emb, Wa, Wv, Wo)


_MESH = plsc.VectorSubcoreMesh(
    core_axis_name="c", subcore_axis_name="s", num_cores=1, num_subcores=_NW
)


def _sc_body(s_hbm, t_hbm, ids_hbm, cpb_hbm, bo_hbm,
             r_hbm, den_hbm, refl_hbm,
             sv, tv, iv, num_acc, den_acc, rtab, tmp, accn, accd,
             cvec, bvec, shn, shd, shr):
    wid = lax.axis_index("s")
    base = wid * _CHUNK

    pltpu.sync_copy(s_hbm.at[pl.ds(base, _CHUNK)], sv)
    pltpu.sync_copy(t_hbm.at[pl.ds(base, _CHUNK)], tv)
    pltpu.sync_copy(ids_hbm.at[pl.ds(base, _CHUNK)], iv)
    pltpu.sync_copy(cpb_hbm, cvec)
    pltpu.sync_copy(bo_hbm, bvec)

    zeros = jnp.zeros((_L,), jnp.float32)

    def zbody(j, c):
        sl = pl.ds(j * _L, _L)
        num_acc[sl] = zeros
        den_acc[sl] = zeros
        return c

    lax.fori_loop(0, _NSEGP // _L, zbody, 0)

    def sbody(i, c):
        sl = pl.ds(i * _L, _L)
        ex = jnp.exp(sv[sl])
        idx = iv[sl]
        plsc.addupdate_scatter(den_acc, [idx], ex)
        plsc.addupdate_scatter(num_acc, [idx], ex * tv[sl])
        return c

    lax.fori_loop(0, _CHUNK // _L, sbody, 0)

    pltpu.sync_copy(num_acc, shn.at[wid])
    pltpu.sync_copy(den_acc, shd.at[wid])
    plsc.subcore_barrier()

    off = wid * _SSEG

    def addloop(j, c):
        sl = pl.ds(j * _L, _L)
        accn[sl] = accn[sl] + tmp[sl]
        return c

    def addloop_d(j, c):
        sl = pl.ds(j * _L, _L)
        accd[sl] = accd[sl] + tmp[sl]
        return c

    pltpu.sync_copy(shn.at[0, pl.ds(off, _SSEG)], accn)
    pltpu.sync_copy(shd.at[0, pl.ds(off, _SSEG)], accd)
    for k in range(1, _NW):
        pltpu.sync_copy(shn.at[k, pl.ds(off, _SSEG)], tmp)
        lax.fori_loop(0, _SSEG // _L, addloop, 0)
        pltpu.sync_copy(shd.at[k, pl.ds(off, _SSEG)], tmp)
        lax.fori_loop(0, _SSEG // _L, addloop_d, 0)

    def fbody(j, c):
        sl = pl.ds(j * _L, _L)
        den = accd[sl]
        r = jnp.where(den > 0.0, accn[sl] / den + cvec[...], bvec[...])
        accn[sl] = r
        return c

    lax.fori_loop(0, _SSEG // _L, fbody, 0)

    pltpu.sync_copy(accn, r_hbm.at[pl.ds(off, _SSEG)])
    pltpu.sync_copy(accd, den_hbm.at[pl.ds(off, _SSEG)])
    pltpu.sync_copy(accn, shr.at[pl.ds(off, _SSEG)])
    plsc.subcore_barrier()

    pltpu.sync_copy(shr, rtab)

    def gbody(i, c):
        sl = pl.ds(i * _L, _L)
        sv[sl] = plsc.load_gather(rtab, [iv[sl]])
        return c

    lax.fori_loop(0, _CHUNK // _L, gbody, 0)
    pltpu.sync_copy(sv, refl_hbm.at[pl.ds(base, _CHUNK)])


_sc_stage = functools.partial(
    pl.kernel,
    out_type=[
        jax.ShapeDtypeStruct((_NSEGP,), jnp.float32),
        jax.ShapeDtypeStruct((_NSEGP,), jnp.float32),
        jax.ShapeDtypeStruct((_B,), jnp.float32),
    ],
    mesh=_MESH,
    compiler_params=pltpu.CompilerParams(needs_layout_passes=False),
    scratch_types=[
        pltpu.VMEM((_CHUNK,), jnp.float32),
        pltpu.VMEM((_CHUNK,), jnp.float32),
        pltpu.VMEM((_CHUNK,), jnp.int32),
        pltpu.VMEM((_NSEGP,), jnp.float32),
        pltpu.VMEM((_NSEGP,), jnp.float32),
        pltpu.VMEM((_NSEGP,), jnp.float32),
        pltpu.VMEM((_SSEG,), jnp.float32),
        pltpu.VMEM((_SSEG,), jnp.float32),
        pltpu.VMEM((_SSEG,), jnp.float32),
        pltpu.VMEM((_L,), jnp.float32),
        pltpu.VMEM((_L,), jnp.float32),
        pltpu.VMEM_SHARED((_NW, _NSEGP), jnp.float32),
        pltpu.VMEM_SHARED((_NW, _NSEGP), jnp.float32),
        pltpu.VMEM_SHARED((_NSEGP,), jnp.float32),
    ],
)(_sc_body)


def kernel(emb, img_ids, Wa, ba, Wv, bv, Wo, bo):
    del ba
    ids = img_ids.astype(jnp.int32)
    scores, t = _tc_stage(emb, Wa, Wv, Wo)
    cpb = jnp.dot(Wo[0], bv) + bo[0]
    cpb_vec = jnp.full((_L,), cpb, jnp.float32)
    bo_vec = jnp.full((_L,), bo[0], jnp.float32)
    r_pad, den_pad, refl = _sc_stage(scores, t, ids, cpb_vec, bo_vec)
    r_flat = r_pad[:_NSEG]
    den = den_pad[:_NSEG]
    occ = den > 0
    n = occ.sum()
    nz = jnp.nonzero(occ, size=_NSEG, fill_value=0)[0]
    unique_ids = jnp.where(jnp.arange(_NSEG) < n, nz, ids[0]).astype(img_ids.dtype)
    return refl[:, None], r_flat[:, None], unique_ids


# speedup vs baseline: 43.1906x; 1.1749x over previous
import functools

import jax
import jax.numpy as jnp
from jax import lax
from jax.experimental import pallas as pl
from jax.experimental.pallas import tpu as pltpu
from jax.experimental.pallas import tpu_sc as plsc

_B = 320000
_E = 128
_NSEG = 10000
_L = 16
_NW = 16
_CHUNK = _B // _NW
_SSEG = 640
_NSEGP = _NW * _SSEG
_TILE = 8192


def _tc_body(emb_ref, wa_ref, wv_ref, wo_ref, s_ref, t_ref):
    e = emb_ref[...]
    wa = wa_ref[0, :]
    weff = jnp.sum(wo_ref[0, :][:, None] * wv_ref[...], axis=0)
    row = jax.lax.broadcasted_iota(jnp.int32, (8, _E), 0)
    k = jnp.where(row == 0, wa[None, :],
                  jnp.where(row == 1, weff[None, :], 0.0))
    st8 = jax.lax.dot_general(
        k, e, (((1,), (1,)), ((), ())),
        preferred_element_type=jnp.float32)
    s_ref[...] = st8[0, :]
    t_ref[...] = st8[1, :]


def _tc_stage(emb, Wa, Wv, Wo):
    n = pl.cdiv(_B, _TILE)
    return pl.pallas_call(
        _tc_body,
        grid=(n,),
        in_specs=[
            pl.BlockSpec((_TILE, _E), lambda i: (i, 0)),
            pl.BlockSpec((1, _E), lambda i: (0, 0)),
            pl.BlockSpec((_E, _E), lambda i: (0, 0)),
            pl.BlockSpec((1, _E), lambda i: (0, 0)),
        ],
        out_specs=[
            pl.BlockSpec((_TILE,), lambda i: (i,)),
            pl.BlockSpec((_TILE,), lambda i: (i,)),
        ],
        out_shape=[
            jax.ShapeDtypeStruct((_B,), jnp.float32),
            jax.ShapeDtypeStruct((_B,), jnp.float32),
        ],
    )(emb, Wa, Wv, Wo)


_MESH = plsc.VectorSubcoreMesh(
    core_axis_name="c", subcore_axis_name="s", num_cores=1, num_subcores=_NW
)


def _sc_body(s_hbm, t_hbm, ids_hbm, cpb_hbm, bo_hbm,
             r_hbm, den_hbm, refl_hbm,
             sv, tv, iv, num_acc, den_acc, rtab, tmp, accn, accd,
             cvec, bvec, shn, shd, shr):
    wid = lax.axis_index("s")
    base = wid * _CHUNK

    pltpu.sync_copy(s_hbm.at[pl.ds(base, _CHUNK)], sv)
    pltpu.sync_copy(t_hbm.at[pl.ds(base, _CHUNK)], tv)
    pltpu.sync_copy(ids_hbm.at[pl.ds(base, _CHUNK)], iv)
    pltpu.sync_copy(cpb_hbm, cvec)
    pltpu.sync_copy(bo_hbm, bvec)

    zeros = jnp.zeros((_L,), jnp.float32)

    @plsc.parallel_loop(0, _NSEGP // _L, unroll=8)
    def zbody(j):
        sl = pl.ds(j * _L, _L)
        num_acc[sl] = zeros
        den_acc[sl] = zeros

    @plsc.parallel_loop(0, _CHUNK // _L, unroll=4)
    def sbody(i):
        sl = pl.ds(i * _L, _L)
        ex = jnp.exp(sv[sl])
        idx = iv[sl]
        plsc.addupdate_scatter(den_acc, [idx], ex)
        plsc.addupdate_scatter(num_acc, [idx], ex * tv[sl])

    pltpu.sync_copy(num_acc, shn.at[wid])
    pltpu.sync_copy(den_acc, shd.at[wid])
    plsc.subcore_barrier()

    off = wid * _SSEG

    pltpu.sync_copy(shn.at[0, pl.ds(off, _SSEG)], accn)
    pltpu.sync_copy(shd.at[0, pl.ds(off, _SSEG)], accd)
    for k in range(1, _NW):
        pltpu.sync_copy(shn.at[k, pl.ds(off, _SSEG)], tmp)

        @plsc.parallel_loop(0, _SSEG // _L, unroll=8)
        def addloop(j):
            sl = pl.ds(j * _L, _L)
            accn[sl] = accn[sl] + tmp[sl]

        pltpu.sync_copy(shd.at[k, pl.ds(off, _SSEG)], tmp)

        @plsc.parallel_loop(0, _SSEG // _L, unroll=8)
        def addloop_d(j):
            sl = pl.ds(j * _L, _L)
            accd[sl] = accd[sl] + tmp[sl]

    @plsc.parallel_loop(0, _SSEG // _L, unroll=8)
    def fbody(j):
        sl = pl.ds(j * _L, _L)
        den = accd[sl]
        r = jnp.where(den > 0.0, accn[sl] / den + cvec[...], bvec[...])
        accn[sl] = r

    pltpu.sync_copy(accn, r_hbm.at[pl.ds(off, _SSEG)])
    pltpu.sync_copy(accd, den_hbm.at[pl.ds(off, _SSEG)])
    pltpu.sync_copy(accn, shr.at[pl.ds(off, _SSEG)])
    plsc.subcore_barrier()

    pltpu.sync_copy(shr, rtab)

    @plsc.parallel_loop(0, _CHUNK // _L, unroll=4)
    def gbody(i):
        sl = pl.ds(i * _L, _L)
        sv[sl] = plsc.load_gather(rtab, [iv[sl]])
    pltpu.sync_copy(sv, refl_hbm.at[pl.ds(base, _CHUNK)])


_sc_stage = functools.partial(
    pl.kernel,
    out_type=[
        jax.ShapeDtypeStruct((_NSEGP,), jnp.float32),
        jax.ShapeDtypeStruct((_NSEGP,), jnp.float32),
        jax.ShapeDtypeStruct((_B,), jnp.float32),
    ],
    mesh=_MESH,
    compiler_params=pltpu.CompilerParams(needs_layout_passes=False),
    scratch_types=[
        pltpu.VMEM((_CHUNK,), jnp.float32),
        pltpu.VMEM((_CHUNK,), jnp.float32),
        pltpu.VMEM((_CHUNK,), jnp.int32),
        pltpu.VMEM((_NSEGP,), jnp.float32),
        pltpu.VMEM((_NSEGP,), jnp.float32),
        pltpu.VMEM((_NSEGP,), jnp.float32),
        pltpu.VMEM((_SSEG,), jnp.float32),
        pltpu.VMEM((_SSEG,), jnp.float32),
        pltpu.VMEM((_SSEG,), jnp.float32),
        pltpu.VMEM((_L,), jnp.float32),
        pltpu.VMEM((_L,), jnp.float32),
        pltpu.VMEM_SHARED((_NW, _NSEGP), jnp.float32),
        pltpu.VMEM_SHARED((_NW, _NSEGP), jnp.float32),
        pltpu.VMEM_SHARED((_NSEGP,), jnp.float32),
    ],
)(_sc_body)


def kernel(emb, img_ids, Wa, ba, Wv, bv, Wo, bo):
    del ba
    ids = img_ids.astype(jnp.int32)
    scores, t = _tc_stage(emb, Wa, Wv, Wo)
    cpb = jnp.dot(Wo[0], bv) + bo[0]
    cpb_vec = jnp.full((_L,), cpb, jnp.float32)
    bo_vec = jnp.full((_L,), bo[0], jnp.float32)
    r_pad, den_pad, refl = _sc_stage(scores, t, ids, cpb_vec, bo_vec)
    r_flat = r_pad[:_NSEG]
    den = den_pad[:_NSEG]
    occ = den > 0
    n = occ.sum()
    nz = jnp.nonzero(occ, size=_NSEG, fill_value=0)[0]
    unique_ids = jnp.where(jnp.arange(_NSEG) < n, nz, ids[0]).astype(img_ids.dtype)
    return refl[:, None], r_flat[:, None], unique_ids
